# baseline (speedup 1.0000x reference)
import functools

import jax
import jax.numpy as jnp
from jax import lax
from jax.experimental import pallas as pl
from jax.experimental.pallas import tpu as pltpu
from jax.experimental.pallas import tpu_sc as plsc

NC = 2
NS = 16
NW = NC * NS
L = 16
CHUNK = 128
GRP = 8
CORE0_FRAC = 0.95
NEG_SLOPE = 0.2


def _pre_alpha_body(x_ref, w_ref, asr, adr, s_ref, d_ref):
    wa_s = jnp.sum(w_ref[...] * asr[...], axis=1)
    wa_d = jnp.sum(w_ref[...] * adr[...], axis=1)
    x = x_ref[...]
    s_ref[...] = jnp.sum(x * wa_s[None, :], axis=1)[None, None, :]
    d_ref[...] = jnp.sum(x * wa_d[None, :], axis=1)[None, None, :]


def _pre_h_body(x_ref, w_ref, h_ref):
    h_ref[...] = jnp.dot(x_ref[...], w_ref[...],
                         preferred_element_type=jnp.float32)


def _post_body(w_ref, dn_ref, b_ref, o_ref):
    w = w_ref[0] + w_ref[1]
    dn = jnp.sum(dn_ref[0], axis=0)
    o_ref[...] = w / (dn[:, None] + 1e-16) + b_ref[...]


def _worker_range(ec0, ec1):
    cid = lax.axis_index("c")
    sid = lax.axis_index("s")
    wid = sid * NC + cid
    ecw = jnp.where(cid == 0, ec0, ec1)
    base = jnp.where(cid == 0, sid * ec0, NS * ec0 + sid * ec1)
    return cid, sid, wid, ecw, base


def _sc_edge_body(n, npad, e_valid, ec0, ec1, als_hbm, ald_hbm, src_hbm,
                  dst_hbm, ee_hbm, dn_hbm,
                  as_v, ad_v, src_v, dst_v, ee_v, dn_v, tmp_v):
    cid, sid, wid, ecw, base = _worker_range(ec0, ec1)
    ecmax = src_v.shape[0]

    pltpu.sync_copy(als_hbm, as_v)
    pltpu.sync_copy(ald_hbm, ad_v)
    pltpu.sync_copy(src_hbm.at[pl.ds(base, ecmax)], src_v)
    pltpu.sync_copy(dst_hbm.at[pl.ds(base, ecmax)], dst_v)

    zeros = jnp.zeros((L,), jnp.float32)

    def zdn(i, _):
        dn_v[pl.ds(i * L, L)] = zeros
        return 0
    lax.fori_loop(0, npad // L, zdn, 0)

    def mx(i, m):
        ms, md = m
        return (jnp.maximum(ms, as_v[pl.ds(i * L, L)]),
                jnp.maximum(md, ad_v[pl.ds(i * L, L)]))
    m0 = jnp.full((L,), -1e30, jnp.float32)
    ms, md = lax.fori_loop(0, n // L, mx, (m0, m0))

    lanes0 = lax.iota(jnp.int32, L)

    def lane_max(v):
        for s in (1, 2, 4, 8):
            tmp_v[...] = v
            v = jnp.maximum(v, plsc.load_gather(tmp_v, [lanes0 ^ s]))
        return v

    cpre = lane_max(ms) + lane_max(md)
    cshift = jnp.where(cpre >= 0.0, cpre, NEG_SLOPE * cpre)

    def stage_a(r, _):
        for c in range(CHUNK // L):
            sv = src_v[r, pl.ds(c * L, L)]
            dv = dst_v[r, pl.ds(c * L, L)]
            e = plsc.load_gather(as_v, [sv]) + plsc.load_gather(ad_v, [dv])
            e = jnp.where(e >= 0.0, e, NEG_SLOPE * e)
            ee = jnp.exp(e - cshift)
            gid = (base + r) * CHUNK + c * L + lanes0
            ee = jnp.where(gid < e_valid, ee, 0.0)
            ee_v[r, pl.ds(c * L, L)] = ee
            plsc.addupdate_scatter(dn_v, [dv], ee)
        return 0
    lax.fori_loop(0, ecw, stage_a, 0)

    def wr_ee(g, _):
        pltpu.sync_copy(ee_v.at[pl.ds(g * GRP, GRP)],
                        ee_hbm.at[pl.ds(base + g * GRP, GRP)])
        return 0
    lax.fori_loop(0, ecw // GRP, wr_ee, 0)
    pltpu.sync_copy(dn_v, dn_hbm.at[wid])


def _sc_msg_body(npad, ec0, ec1, h_hbm, src_hbm, dst_hbm, ee_hbm, wsum_hbm,
                 src_v, dst_v, ee_v, rows0_v, rows1_v, wsum_s, sem0, sem1,
                 ssem0, ssem1):
    cid, sid, wid, ecw, base = _worker_range(ec0, ec1)
    nt = npad // NS
    d = rows0_v.shape[1]
    nj = d // L
    rows = (rows0_v, rows1_v)
    sems = (sem0, sem1)
    ssems = (ssem0, ssem1)

    zeros = jnp.zeros((L,), jnp.float32)

    def zrows(k, _):
        for j in range(nj):
            rows0_v[k, pl.ds(j * L, L)] = zeros
        return 0
    lax.fori_loop(0, CHUNK, zrows, 0)
    for b in range(nt // CHUNK):
        pltpu.sync_copy(rows0_v,
                        wsum_s.at[pl.ds(sid * nt + b * CHUNK, CHUNK)])

    plsc.subcore_barrier()

    def group(g, _):
        gb = base + g * GRP
        pltpu.sync_copy(src_hbm.at[pl.ds(gb, GRP)], src_v)
        pltpu.sync_copy(dst_hbm.at[pl.ds(gb, GRP)], dst_v)
        pltpu.sync_copy(ee_hbm.at[pl.ds(gb, GRP)], ee_v)

        pltpu.async_copy(h_hbm.at[src_v.at[0]], rows[0], sems[0])
        for j in range(GRP):
            b = j & 1
            pltpu.make_async_copy(h_hbm.at[src_v.at[j]], rows[b],
                                  sems[b]).wait()

            @plsc.parallel_loop(0, CHUNK, unroll=4)
            def scale(k):
                cvec = plsc.load_gather(
                    ee_v, [jnp.full((L,), j, jnp.int32),
                           jnp.full((L,), k, jnp.int32)])
                for jj in range(nj):
                    rows[b][k, pl.ds(jj * L, L)] = (
                        rows[b][k, pl.ds(jj * L, L)] * cvec)

            pltpu.make_async_copy(rows[b], wsum_s.at[dst_v.at[j]],
                                  ssems[b]).start(add=True)
            if j + 1 < GRP:
                if j >= 1:
                    pltpu.make_async_copy(rows[1 - b],
                                          wsum_s.at[dst_v.at[j - 1]],
                                          ssems[1 - b]).wait()
                pltpu.async_copy(h_hbm.at[src_v.at[j + 1]], rows[1 - b],
                                 sems[1 - b])
        pltpu.make_async_copy(rows[0], wsum_s.at[dst_v.at[0]],
                              ssems[0]).wait()
        pltpu.make_async_copy(rows[1], wsum_s.at[dst_v.at[0]],
                              ssems[1]).wait()
        return 0
    lax.fori_loop(0, ecw // GRP, group, 0)

    plsc.subcore_barrier()

    pltpu.sync_copy(wsum_s.at[pl.ds(sid * nt, nt)],
                    wsum_hbm.at[cid].at[pl.ds(sid * nt, nt)])


def kernel(x, edge_index, W, a_src, a_dst, bias):
    n, d = x.shape
    e = edge_index.shape[1]

    x = x.astype(jnp.float32)
    W = W.astype(jnp.float32)
    a_src = a_src.astype(jnp.float32)
    a_dst = a_dst.astype(jnp.float32)
    bias = bias.astype(jnp.float32)

    ecavg = -(-e // (NW * CHUNK * (GRP // NC))) * (GRP // NC)
    ec0 = max(GRP, int(round(CORE0_FRAC * NC * ecavg / GRP)) * GRP)
    ec1 = NC * ecavg - ec0
    ecmax = max(ec0, ec1)
    totc = NS * (ec0 + ec1)
    totc_in = totc + ecmax
    epad = totc_in * CHUNK
    npad = -(-n // (NS * CHUNK)) * NS * CHUNK
    src = edge_index[0].astype(jnp.int32)
    dst = edge_index[1].astype(jnp.int32)
    pad = epad - e
    zpad = jnp.zeros((pad,), jnp.int32)
    src_p = jnp.concatenate([src, zpad]).reshape(totc_in, CHUNK)
    dst_p = jnp.concatenate([dst, zpad]).reshape(totc_in, CHUNK)

    bn = 400
    nb = n // bn
    s3, d3 = pl.pallas_call(
        _pre_alpha_body,
        grid=(nb,),
        in_specs=[
            pl.BlockSpec((bn, d), lambda i: (i, 0)),
            pl.BlockSpec((d, d), lambda i: (0, 0)),
            pl.BlockSpec((1, d), lambda i: (0, 0)),
            pl.BlockSpec((1, d), lambda i: (0, 0)),
        ],
        out_specs=[
            pl.BlockSpec((1, 1, bn), lambda i: (i, 0, 0)),
            pl.BlockSpec((1, 1, bn), lambda i: (i, 0, 0)),
        ],
        out_shape=[
            jax.ShapeDtypeStruct((nb, 1, bn), jnp.float32),
            jax.ShapeDtypeStruct((nb, 1, bn), jnp.float32),
        ],
    )(x, W, a_src.reshape(1, d), a_dst.reshape(1, d))
    als = s3.reshape(n)
    ald = d3.reshape(n)
    h = pl.pallas_call(
        _pre_h_body,
        grid=(nb,),
        in_specs=[
            pl.BlockSpec((bn, d), lambda i: (i, 0)),
            pl.BlockSpec((d, d), lambda i: (0, 0)),
        ],
        out_specs=pl.BlockSpec((bn, d), lambda i: (i, 0)),
        out_shape=jax.ShapeDtypeStruct((n, d), jnp.float32),
    )(x, W)

    mesh = plsc.VectorSubcoreMesh(core_axis_name="c", subcore_axis_name="s",
                                  num_cores=NC, num_subcores=NS)
    sc_params = pltpu.CompilerParams(needs_layout_passes=False)

    eew, dnp = pl.kernel(
        functools.partial(_sc_edge_body, n, npad, e, ec0, ec1),
        out_type=[
            jax.ShapeDtypeStruct((totc, CHUNK), jnp.float32),
            jax.ShapeDtypeStruct((NW, npad), jnp.float32),
        ],
        mesh=mesh,
        compiler_params=sc_params,
        scratch_types=[
            pltpu.VMEM((n,), jnp.float32),
            pltpu.VMEM((n,), jnp.float32),
            pltpu.VMEM((ecmax, CHUNK), jnp.int32),
            pltpu.VMEM((ecmax, CHUNK), jnp.int32),
            pltpu.VMEM((ecmax, CHUNK), jnp.float32),
            pltpu.VMEM((npad,), jnp.float32),
            pltpu.VMEM((L,), jnp.float32),
        ],
    )(als, ald, src_p, dst_p)

    wsum = pl.kernel(
        functools.partial(_sc_msg_body, npad, ec0, ec1),
        out_type=jax.ShapeDtypeStruct((NC, npad, d), jnp.float32),
        mesh=mesh,
        compiler_params=sc_params,
        scratch_types=[
            pltpu.VMEM((GRP, CHUNK), jnp.int32),
            pltpu.VMEM((GRP, CHUNK), jnp.int32),
            pltpu.VMEM((GRP, CHUNK), jnp.float32),
            pltpu.VMEM((CHUNK, d), jnp.float32),
            pltpu.VMEM((CHUNK, d), jnp.float32),
            pltpu.VMEM_SHARED((npad, d), jnp.float32),
            pltpu.SemaphoreType.DMA,
            pltpu.SemaphoreType.DMA,
            pltpu.SemaphoreType.DMA,
            pltpu.SemaphoreType.DMA,
        ],
    )(h, src_p, dst_p, eew)

    dn3 = dnp[:, :n].reshape(NW, nb, bn).transpose(1, 0, 2)
    out = pl.pallas_call(
        _post_body,
        grid=(nb,),
        in_specs=[
            pl.BlockSpec((NC, bn, d), lambda i: (0, i, 0)),
            pl.BlockSpec((1, NW, bn), lambda i: (i, 0, 0)),
            pl.BlockSpec((1, d), lambda i: (0, 0)),
        ],
        out_specs=pl.BlockSpec((bn, d), lambda i: (i, 0)),
        out_shape=jax.ShapeDtypeStruct((n, d), jnp.float32),
    )(wsum, dn3, bias.reshape(1, d))
    return out


# speedup vs baseline: 1.1847x; 1.1847x over previous
import functools

import jax
import jax.numpy as jnp
from jax import lax
from jax.experimental import pallas as pl
from jax.experimental.pallas import tpu as pltpu
from jax.experimental.pallas import tpu_sc as plsc

NC = 2
NS = 16
NW = NC * NS
L = 16
CHUNK = 128
GRP = 16
CORE0_FRAC = 0.9
NEG_SLOPE = 0.2


def _pre_alpha_body(x_ref, w_ref, asr, adr, s_ref, d_ref):
    wa_s = jnp.sum(w_ref[...] * asr[...], axis=1)
    wa_d = jnp.sum(w_ref[...] * adr[...], axis=1)
    x = x_ref[...]
    s_ref[...] = jnp.sum(x * wa_s[None, :], axis=1)[None, None, :]
    d_ref[...] = jnp.sum(x * wa_d[None, :], axis=1)[None, None, :]


def _pre_h_body(x_ref, w_ref, h_ref):
    h_ref[...] = jnp.dot(x_ref[...], w_ref[...],
                         preferred_element_type=jnp.float32)


def _post_body(w_ref, dn_ref, b_ref, o_ref):
    w = w_ref[0] + w_ref[1]
    dn = jnp.sum(dn_ref[0], axis=0)
    o_ref[...] = w / (dn[:, None] + 1e-16) + b_ref[...]


def _worker_range(ec0, ec1):
    cid = lax.axis_index("c")
    sid = lax.axis_index("s")
    wid = sid * NC + cid
    ecw = jnp.where(cid == 0, ec0, ec1)
    base = jnp.where(cid == 0, sid * ec0, NS * ec0 + sid * ec1)
    return cid, sid, wid, ecw, base


def _sc_edge_body(n, npad, e_valid, ec0, ec1, als_hbm, ald_hbm, src_hbm,
                  dst_hbm, ee_hbm, dn_hbm,
                  as_v, ad_v, src_v, dst_v, ee_v, dn_v, tmp_v):
    cid, sid, wid, ecw, base = _worker_range(ec0, ec1)
    ecmax = src_v.shape[0]

    pltpu.sync_copy(als_hbm, as_v)
    pltpu.sync_copy(ald_hbm, ad_v)
    pltpu.sync_copy(src_hbm.at[pl.ds(base, ecmax)], src_v)
    pltpu.sync_copy(dst_hbm.at[pl.ds(base, ecmax)], dst_v)

    zeros = jnp.zeros((L,), jnp.float32)

    def zdn(i, _):
        dn_v[pl.ds(i * L, L)] = zeros
        return 0
    lax.fori_loop(0, npad // L, zdn, 0)

    def mx(i, m):
        ms, md = m
        return (jnp.maximum(ms, as_v[pl.ds(i * L, L)]),
                jnp.maximum(md, ad_v[pl.ds(i * L, L)]))
    m0 = jnp.full((L,), -1e30, jnp.float32)
    ms, md = lax.fori_loop(0, n // L, mx, (m0, m0))

    lanes0 = lax.iota(jnp.int32, L)

    def lane_max(v):
        for s in (1, 2, 4, 8):
            tmp_v[...] = v
            v = jnp.maximum(v, plsc.load_gather(tmp_v, [lanes0 ^ s]))
        return v

    cpre = lane_max(ms) + lane_max(md)
    cshift = jnp.where(cpre >= 0.0, cpre, NEG_SLOPE * cpre)

    def stage_a(r, _):
        for c in range(CHUNK // L):
            sv = src_v[r, pl.ds(c * L, L)]
            dv = dst_v[r, pl.ds(c * L, L)]
            e = plsc.load_gather(as_v, [sv]) + plsc.load_gather(ad_v, [dv])
            e = jnp.where(e >= 0.0, e, NEG_SLOPE * e)
            ee = jnp.exp(e - cshift)
            gid = (base + r) * CHUNK + c * L + lanes0
            ee = jnp.where(gid < e_valid, ee, 0.0)
            ee_v[r, pl.ds(c * L, L)] = ee
            plsc.addupdate_scatter(dn_v, [dv], ee)
        return 0
    lax.fori_loop(0, ecw, stage_a, 0)

    def wr_ee(g, _):
        pltpu.sync_copy(ee_v.at[pl.ds(g * GRP, GRP)],
                        ee_hbm.at[pl.ds(base + g * GRP, GRP)])
        return 0
    lax.fori_loop(0, ecw // GRP, wr_ee, 0)
    pltpu.sync_copy(dn_v, dn_hbm.at[wid])


def _sc_msg_body(npad, ec0, ec1, h_hbm, src_hbm, dst_hbm, ee_hbm, wsum_hbm,
                 src_v, dst_v, ee_v, rows0_v, rows1_v, wsum_s, sem0, sem1,
                 ssem0, ssem1):
    cid, sid, wid, ecw, base = _worker_range(ec0, ec1)
    nt = npad // NS
    d = rows0_v.shape[1]
    nj = d // L
    rows = (rows0_v, rows1_v)
    sems = (sem0, sem1)
    ssems = (ssem0, ssem1)

    zeros = jnp.zeros((L,), jnp.float32)

    def zrows(k, _):
        for j in range(nj):
            rows0_v[k, pl.ds(j * L, L)] = zeros
        return 0
    lax.fori_loop(0, CHUNK, zrows, 0)
    for b in range(nt // CHUNK):
        pltpu.sync_copy(rows0_v,
                        wsum_s.at[pl.ds(sid * nt + b * CHUNK, CHUNK)])

    plsc.subcore_barrier()

    def group(g, _):
        gb = base + g * GRP
        pltpu.sync_copy(src_hbm.at[pl.ds(gb, GRP)], src_v)
        pltpu.sync_copy(dst_hbm.at[pl.ds(gb, GRP)], dst_v)
        pltpu.sync_copy(ee_hbm.at[pl.ds(gb, GRP)], ee_v)

        pltpu.async_copy(h_hbm.at[src_v.at[0]], rows[0], sems[0])
        for j in range(GRP):
            b = j & 1
            pltpu.make_async_copy(h_hbm.at[src_v.at[j]], rows[b],
                                  sems[b]).wait()
            if j + 1 < GRP:
                if j >= 1:
                    pltpu.make_async_copy(rows[1 - b],
                                          wsum_s.at[dst_v.at[j - 1]],
                                          ssems[1 - b]).wait()
                pltpu.async_copy(h_hbm.at[src_v.at[j + 1]], rows[1 - b],
                                 sems[1 - b])

            @plsc.parallel_loop(0, CHUNK, unroll=4)
            def scale(k):
                cvec = plsc.load_gather(
                    ee_v, [jnp.full((L,), j, jnp.int32),
                           jnp.full((L,), k, jnp.int32)])
                for jj in range(nj):
                    rows[b][k, pl.ds(jj * L, L)] = (
                        rows[b][k, pl.ds(jj * L, L)] * cvec)

            pltpu.make_async_copy(rows[b], wsum_s.at[dst_v.at[j]],
                                  ssems[b]).start(add=True)
        pltpu.make_async_copy(rows[0], wsum_s.at[dst_v.at[0]],
                              ssems[0]).wait()
        pltpu.make_async_copy(rows[1], wsum_s.at[dst_v.at[0]],
                              ssems[1]).wait()
        return 0
    lax.fori_loop(0, ecw // GRP, group, 0)

    plsc.subcore_barrier()

    pltpu.sync_copy(wsum_s.at[pl.ds(sid * nt, nt)],
                    wsum_hbm.at[cid].at[pl.ds(sid * nt, nt)])


def kernel(x, edge_index, W, a_src, a_dst, bias):
    n, d = x.shape
    e = edge_index.shape[1]

    x = x.astype(jnp.float32)
    W = W.astype(jnp.float32)
    a_src = a_src.astype(jnp.float32)
    a_dst = a_dst.astype(jnp.float32)
    bias = bias.astype(jnp.float32)

    ecavg = -(-e // (NW * CHUNK * (GRP // NC))) * (GRP // NC)
    ec0 = max(GRP, int(round(CORE0_FRAC * NC * ecavg / GRP)) * GRP)
    ec1 = NC * ecavg - ec0
    ecmax = max(ec0, ec1)
    totc = NS * (ec0 + ec1)
    totc_in = totc + ecmax
    epad = totc_in * CHUNK
    npad = -(-n // (NS * CHUNK)) * NS * CHUNK
    src = edge_index[0].astype(jnp.int32)
    dst = edge_index[1].astype(jnp.int32)
    pad = epad - e
    zpad = jnp.zeros((pad,), jnp.int32)
    src_p = jnp.concatenate([src, zpad]).reshape(totc_in, CHUNK)
    dst_p = jnp.concatenate([dst, zpad]).reshape(totc_in, CHUNK)

    bn = 400
    nb = n // bn
    s3, d3 = pl.pallas_call(
        _pre_alpha_body,
        grid=(nb,),
        in_specs=[
            pl.BlockSpec((bn, d), lambda i: (i, 0)),
            pl.BlockSpec((d, d), lambda i: (0, 0)),
            pl.BlockSpec((1, d), lambda i: (0, 0)),
            pl.BlockSpec((1, d), lambda i: (0, 0)),
        ],
        out_specs=[
            pl.BlockSpec((1, 1, bn), lambda i: (i, 0, 0)),
            pl.BlockSpec((1, 1, bn), lambda i: (i, 0, 0)),
        ],
        out_shape=[
            jax.ShapeDtypeStruct((nb, 1, bn), jnp.float32),
            jax.ShapeDtypeStruct((nb, 1, bn), jnp.float32),
        ],
    )(x, W, a_src.reshape(1, d), a_dst.reshape(1, d))
    als = s3.reshape(n)
    ald = d3.reshape(n)
    h = pl.pallas_call(
        _pre_h_body,
        grid=(nb,),
        in_specs=[
            pl.BlockSpec((bn, d), lambda i: (i, 0)),
            pl.BlockSpec((d, d), lambda i: (0, 0)),
        ],
        out_specs=pl.BlockSpec((bn, d), lambda i: (i, 0)),
        out_shape=jax.ShapeDtypeStruct((n, d), jnp.float32),
    )(x, W)

    mesh = plsc.VectorSubcoreMesh(core_axis_name="c", subcore_axis_name="s",
                                  num_cores=NC, num_subcores=NS)
    sc_params = pltpu.CompilerParams(needs_layout_passes=False)

    eew, dnp = pl.kernel(
        functools.partial(_sc_edge_body, n, npad, e, ec0, ec1),
        out_type=[
            jax.ShapeDtypeStruct((totc, CHUNK), jnp.float32),
            jax.ShapeDtypeStruct((NW, npad), jnp.float32),
        ],
        mesh=mesh,
        compiler_params=sc_params,
        scratch_types=[
            pltpu.VMEM((n,), jnp.float32),
            pltpu.VMEM((n,), jnp.float32),
            pltpu.VMEM((ecmax, CHUNK), jnp.int32),
            pltpu.VMEM((ecmax, CHUNK), jnp.int32),
            pltpu.VMEM((ecmax, CHUNK), jnp.float32),
            pltpu.VMEM((npad,), jnp.float32),
            pltpu.VMEM((L,), jnp.float32),
        ],
    )(als, ald, src_p, dst_p)

    wsum = pl.kernel(
        functools.partial(_sc_msg_body, npad, ec0, ec1),
        out_type=jax.ShapeDtypeStruct((NC, npad, d), jnp.float32),
        mesh=mesh,
        compiler_params=sc_params,
        scratch_types=[
            pltpu.VMEM((GRP, CHUNK), jnp.int32),
            pltpu.VMEM((GRP, CHUNK), jnp.int32),
            pltpu.VMEM((GRP, CHUNK), jnp.float32),
            pltpu.VMEM((CHUNK, d), jnp.float32),
            pltpu.VMEM((CHUNK, d), jnp.float32),
            pltpu.VMEM_SHARED((npad, d), jnp.float32),
            pltpu.SemaphoreType.DMA,
            pltpu.SemaphoreType.DMA,
            pltpu.SemaphoreType.DMA,
            pltpu.SemaphoreType.DMA,
        ],
    )(h, src_p, dst_p, eew)

    dn3 = dnp[:, :n].reshape(NW, nb, bn).transpose(1, 0, 2)
    out = pl.pallas_call(
        _post_body,
        grid=(nb,),
        in_specs=[
            pl.BlockSpec((NC, bn, d), lambda i: (0, i, 0)),
            pl.BlockSpec((1, NW, bn), lambda i: (i, 0, 0)),
            pl.BlockSpec((1, d), lambda i: (0, 0)),
        ],
        out_specs=pl.BlockSpec((bn, d), lambda i: (i, 0)),
        out_shape=jax.ShapeDtypeStruct((n, d), jnp.float32),
    )(wsum, dn3, bias.reshape(1, d))
    return out


# speedup vs baseline: 1.1968x; 1.0102x over previous
import functools

import jax
import jax.numpy as jnp
from jax import lax
from jax.experimental import pallas as pl
from jax.experimental.pallas import tpu as pltpu
from jax.experimental.pallas import tpu_sc as plsc

NC = 2
NS = 16
NW = NC * NS
L = 16
CHUNK = 128
GRP = 16
CORE0_FRAC = 0.9
NEG_SLOPE = 0.2


def _pre_alpha_body(x_ref, w_ref, asr, adr, s_ref, d_ref):
    wa_s = jnp.sum(w_ref[...] * asr[...], axis=1)
    wa_d = jnp.sum(w_ref[...] * adr[...], axis=1)
    x = x_ref[...]
    s_ref[...] = jnp.sum(x * wa_s[None, :], axis=1)[None, None, :]
    d_ref[...] = jnp.sum(x * wa_d[None, :], axis=1)[None, None, :]


def _pre_h_body(x_ref, w_ref, h_ref):
    h_ref[...] = jnp.dot(x_ref[...], w_ref[...],
                         preferred_element_type=jnp.float32)


def _post_body(w_ref, dn_ref, b_ref, o_ref):
    w = w_ref[0] + w_ref[1]
    dn = jnp.sum(dn_ref[0], axis=0)
    o_ref[...] = w / (dn[:, None] + 1e-16) + b_ref[...]


def _worker_range(ec0, ec1):
    cid = lax.axis_index("c")
    sid = lax.axis_index("s")
    wid = sid * NC + cid
    ecw = jnp.where(cid == 0, ec0, ec1)
    base = jnp.where(cid == 0, sid * ec0, NS * ec0 + sid * ec1)
    return cid, sid, wid, ecw, base


def _sc_edge_body(n, npad, e_valid, ec0, ec1, als_hbm, ald_hbm, src_hbm,
                  dst_hbm, ee_hbm, dn_hbm,
                  as_v, ad_v, src_v, dst_v, ee_v, dn_v, tmp_v):
    cid, sid, wid, ecw, base = _worker_range(ec0, ec1)
    ecmax = src_v.shape[0]

    pltpu.sync_copy(als_hbm, as_v)
    pltpu.sync_copy(ald_hbm, ad_v)
    pltpu.sync_copy(src_hbm.at[pl.ds(base, ecmax)], src_v)
    pltpu.sync_copy(dst_hbm.at[pl.ds(base, ecmax)], dst_v)

    zeros = jnp.zeros((L,), jnp.float32)

    def zdn(i, _):
        dn_v[pl.ds(i * L, L)] = zeros
        return 0
    lax.fori_loop(0, npad // L, zdn, 0)

    def mx(i, m):
        ms, md = m
        return (jnp.maximum(ms, as_v[pl.ds(i * L, L)]),
                jnp.maximum(md, ad_v[pl.ds(i * L, L)]))
    m0 = jnp.full((L,), -1e30, jnp.float32)
    ms, md = lax.fori_loop(0, n // L, mx, (m0, m0))

    lanes0 = lax.iota(jnp.int32, L)

    def lane_max(v):
        for s in (1, 2, 4, 8):
            tmp_v[...] = v
            v = jnp.maximum(v, plsc.load_gather(tmp_v, [lanes0 ^ s]))
        return v

    cpre = lane_max(ms) + lane_max(md)
    cshift = jnp.where(cpre >= 0.0, cpre, NEG_SLOPE * cpre)

    def stage_a(r, _):
        for c in range(CHUNK // L):
            sv = src_v[r, pl.ds(c * L, L)]
            dv = dst_v[r, pl.ds(c * L, L)]
            e = plsc.load_gather(as_v, [sv]) + plsc.load_gather(ad_v, [dv])
            e = jnp.where(e >= 0.0, e, NEG_SLOPE * e)
            ee = jnp.exp(e - cshift)
            gid = (base + r) * CHUNK + c * L + lanes0
            ee = jnp.where(gid < e_valid, ee, 0.0)
            ee_v[r, pl.ds(c * L, L)] = ee
            plsc.addupdate_scatter(dn_v, [dv], ee)
        return 0
    lax.fori_loop(0, ecw, stage_a, 0)

    def wr_ee(g, _):
        pltpu.sync_copy(ee_v.at[pl.ds(g * GRP, GRP)],
                        ee_hbm.at[pl.ds(base + g * GRP, GRP)])
        return 0
    lax.fori_loop(0, ecw // GRP, wr_ee, 0)
    pltpu.sync_copy(dn_v, dn_hbm.at[wid])


def _sc_msg_body(npad, ec0, ec1, h_hbm, src_hbm, dst_hbm, ee_hbm, wsum_hbm,
                 src_v, dst_v, ee_v, rows0_v, rows1_v, wsum_s, sem0, sem1,
                 ssem0, ssem1):
    cid, sid, wid, ecw, base = _worker_range(ec0, ec1)
    nt = npad // NS
    d = rows0_v.shape[1]
    nj = d // L
    rows = (rows0_v, rows1_v)
    sems = (sem0, sem1)
    ssems = (ssem0, ssem1)

    zeros = jnp.zeros((L,), jnp.float32)

    def zrows(k, _):
        for j in range(nj):
            rows0_v[k, pl.ds(j * L, L)] = zeros
        return 0
    lax.fori_loop(0, CHUNK, zrows, 0)
    for b in range(nt // CHUNK):
        pltpu.sync_copy(rows0_v,
                        wsum_s.at[pl.ds(sid * nt + b * CHUNK, CHUNK)])

    plsc.subcore_barrier()

    def group(g, _):
        gb = base + g * GRP
        pltpu.sync_copy(src_hbm.at[pl.ds(gb, GRP)], src_v)
        pltpu.sync_copy(dst_hbm.at[pl.ds(gb, GRP)], dst_v)
        pltpu.sync_copy(ee_hbm.at[pl.ds(gb, GRP)], ee_v)

        pltpu.async_copy(h_hbm.at[src_v.at[0]], rows[0], sems[0])
        for j in range(GRP):
            b = j & 1
            pltpu.make_async_copy(h_hbm.at[src_v.at[j]], rows[b],
                                  sems[b]).wait()
            if j + 1 < GRP:
                if j >= 1:
                    pltpu.make_async_copy(rows[1 - b],
                                          wsum_s.at[dst_v.at[j - 1]],
                                          ssems[1 - b]).wait()
                pltpu.async_copy(h_hbm.at[src_v.at[j + 1]], rows[1 - b],
                                 sems[1 - b])

            @plsc.parallel_loop(0, CHUNK, unroll=4)
            def scale(k):
                cvec = plsc.load_gather(
                    ee_v, [jnp.full((L,), j, jnp.int32),
                           jnp.full((L,), k, jnp.int32)])
                for jj in range(nj):
                    rows[b][k, pl.ds(jj * L, L)] = (
                        rows[b][k, pl.ds(jj * L, L)] * cvec)

            pltpu.make_async_copy(rows[b], wsum_s.at[dst_v.at[j]],
                                  ssems[b]).start(add=True)
        pltpu.make_async_copy(rows[0], wsum_s.at[dst_v.at[0]],
                              ssems[0]).wait()
        pltpu.make_async_copy(rows[1], wsum_s.at[dst_v.at[0]],
                              ssems[1]).wait()
        return 0
    lax.fori_loop(0, ecw // GRP, group, 0)

    plsc.subcore_barrier()

    pltpu.sync_copy(wsum_s.at[pl.ds(sid * nt, nt)],
                    wsum_hbm.at[cid].at[pl.ds(sid * nt, nt)])


def kernel(x, edge_index, W, a_src, a_dst, bias):
    n, d = x.shape
    e = edge_index.shape[1]

    x = x.astype(jnp.float32)
    W = W.astype(jnp.float32)
    a_src = a_src.astype(jnp.float32)
    a_dst = a_dst.astype(jnp.float32)
    bias = bias.astype(jnp.float32)

    ecavg = -(-e // (NW * CHUNK * (GRP // NC))) * (GRP // NC)
    ec0 = max(GRP, int(round(CORE0_FRAC * NC * ecavg / GRP)) * GRP)
    ec1 = NC * ecavg - ec0
    ecmax = max(ec0, ec1)
    totc = NS * (ec0 + ec1)
    totc_in = totc + ecmax
    epad = totc_in * CHUNK
    npad = -(-n // (NS * CHUNK)) * NS * CHUNK
    src = edge_index[0].astype(jnp.int32)
    dst = edge_index[1].astype(jnp.int32)
    pad = epad - e
    zpad = jnp.zeros((pad,), jnp.int32)
    src_p = jnp.concatenate([src, zpad]).reshape(totc_in, CHUNK)
    dst_p = jnp.concatenate([dst, zpad]).reshape(totc_in, CHUNK)

    bn = 400
    nb = n // bn
    s3, d3 = pl.pallas_call(
        _pre_alpha_body,
        grid=(nb,),
        in_specs=[
            pl.BlockSpec((bn, d), lambda i: (i, 0)),
            pl.BlockSpec((d, d), lambda i: (0, 0)),
            pl.BlockSpec((1, d), lambda i: (0, 0)),
            pl.BlockSpec((1, d), lambda i: (0, 0)),
        ],
        out_specs=[
            pl.BlockSpec((1, 1, bn), lambda i: (i, 0, 0)),
            pl.BlockSpec((1, 1, bn), lambda i: (i, 0, 0)),
        ],
        out_shape=[
            jax.ShapeDtypeStruct((nb, 1, bn), jnp.float32),
            jax.ShapeDtypeStruct((nb, 1, bn), jnp.float32),
        ],
    )(x, W, a_src.reshape(1, d), a_dst.reshape(1, d))
    als = s3.reshape(n)
    ald = d3.reshape(n)
    h = pl.pallas_call(
        _pre_h_body,
        grid=(nb,),
        in_specs=[
            pl.BlockSpec((bn, d), lambda i: (i, 0)),
            pl.BlockSpec((d, d), lambda i: (0, 0)),
        ],
        out_specs=pl.BlockSpec((bn, d), lambda i: (i, 0)),
        out_shape=jax.ShapeDtypeStruct((n, d), jnp.float32),
    )(x, W)

    mesh = plsc.VectorSubcoreMesh(core_axis_name="c", subcore_axis_name="s",
                                  num_cores=NC, num_subcores=NS)
    sc_params = pltpu.CompilerParams(needs_layout_passes=False)

    ece = (ec0 + ec1) // 2
    eew, dnp = pl.kernel(
        functools.partial(_sc_edge_body, n, npad, e, ece, ece),
        out_type=[
            jax.ShapeDtypeStruct((totc, CHUNK), jnp.float32),
            jax.ShapeDtypeStruct((NW, npad), jnp.float32),
        ],
        mesh=mesh,
        compiler_params=sc_params,
        scratch_types=[
            pltpu.VMEM((n,), jnp.float32),
            pltpu.VMEM((n,), jnp.float32),
            pltpu.VMEM((ece, CHUNK), jnp.int32),
            pltpu.VMEM((ece, CHUNK), jnp.int32),
            pltpu.VMEM((ece, CHUNK), jnp.float32),
            pltpu.VMEM((npad,), jnp.float32),
            pltpu.VMEM((L,), jnp.float32),
        ],
    )(als, ald, src_p, dst_p)

    wsum = pl.kernel(
        functools.partial(_sc_msg_body, npad, ec0, ec1),
        out_type=jax.ShapeDtypeStruct((NC, npad, d), jnp.float32),
        mesh=mesh,
        compiler_params=sc_params,
        scratch_types=[
            pltpu.VMEM((GRP, CHUNK), jnp.int32),
            pltpu.VMEM((GRP, CHUNK), jnp.int32),
            pltpu.VMEM((GRP, CHUNK), jnp.float32),
            pltpu.VMEM((CHUNK, d), jnp.float32),
            pltpu.VMEM((CHUNK, d), jnp.float32),
            pltpu.VMEM_SHARED((npad, d), jnp.float32),
            pltpu.SemaphoreType.DMA,
            pltpu.SemaphoreType.DMA,
            pltpu.SemaphoreType.DMA,
            pltpu.SemaphoreType.DMA,
        ],
    )(h, src_p, dst_p, eew)

    dn3 = dnp[:, :n].reshape(NW, nb, bn).transpose(1, 0, 2)
    out = pl.pallas_call(
        _post_body,
        grid=(nb,),
        in_specs=[
            pl.BlockSpec((NC, bn, d), lambda i: (0, i, 0)),
            pl.BlockSpec((1, NW, bn), lambda i: (i, 0, 0)),
            pl.BlockSpec((1, d), lambda i: (0, 0)),
        ],
        out_specs=pl.BlockSpec((bn, d), lambda i: (i, 0)),
        out_shape=jax.ShapeDtypeStruct((n, d), jnp.float32),
    )(wsum, dn3, bias.reshape(1, d))
    return out
